# baseline (speedup 1.0000x reference)
import jax
import jax.numpy as jnp
from jax import lax
from jax.experimental import pallas as pl
from jax.experimental.pallas import tpu as pltpu
from jax.experimental.pallas import tpu_sc as plsc

DIM = 256
HALF = DIM // 2
PAD = 144
BATCH = 4096
NPAIRS = 64
NW = 32
L = 16
NG = HALF // L
UNROLL = 8

SC_ROWS = DIM
ROWS_PER_W = SC_ROWS // NW
MM_TILE = 4096


def _trig_body(te_ref, to_ref, ce_ref, se_ref, co_ref, so_ref):
    ce_ref[...] = jnp.cos(2.0 * te_ref[...])
    se_ref[...] = jnp.sin(2.0 * te_ref[...])
    to2 = 2.0 * to_ref[...]
    pad1 = jnp.ones((NPAIRS, 1), jnp.float32)
    pad0 = jnp.zeros((NPAIRS, 1), jnp.float32)
    co_ref[...] = jnp.concatenate([jnp.cos(to2), pad1], axis=1)
    so_ref[...] = jnp.concatenate([jnp.sin(to2), pad0], axis=1)


_trig = pl.pallas_call(
    _trig_body,
    out_shape=[jax.ShapeDtypeStruct((NPAIRS, HALF), jnp.float32)] * 4,
)


def _take(v, idx):
    return lax.gather(
        v, idx[:, None],
        lax.GatherDimensionNumbers(offset_dims=(), collapsed_slice_dims=(0,),
                                   start_index_map=(0,)),
        slice_sizes=(1,),
        mode=lax.GatherScatterMode.PROMISE_IN_BOUNDS)


def _clements_body(ce_hbm, se_hbm, co_hbm, so_hbm, out_hbm,
                   X, A, B, CE, SE, CO, SO, dsem):
    wid = lax.axis_index("s") * 2 + lax.axis_index("c")
    base = wid * ROWS_PER_W
    copies = [pltpu.async_copy(src, dst, dsem) for src, dst in
              ((ce_hbm, CE), (se_hbm, SE), (co_hbm, CO), (so_hbm, SO))]

    iota = lax.iota(jnp.int32, L)
    zeros = jnp.zeros((L,), jnp.float32)
    ones = jnp.ones((L,), jnp.float32)

    jc = base // 2
    for r in range(ROWS_PER_W):
        hot = jnp.full((L,), jc + r // 2, jnp.int32)
        for j in range(NG):
            onehot = jnp.where(iota + L * j == hot, ones, zeros)
            if r % 2 == 0:
                A[r, pl.ds(L * j, L)] = onehot
                B[r, pl.ds(L * j, L)] = zeros
            else:
                A[r, pl.ds(L * j, L)] = zeros
                B[r, pl.ds(L * j, L)] = onehot
        A[r, pl.ds(HALF, L)] = zeros

    for c in copies:
        c.wait()

    def layer_pair2(l2, carry):
        for l in (2 * l2, 2 * l2 + 1):
            _do_pair(l)
        return carry

    def _do_pair(l):
        ces = [CE[l, pl.ds(L * g, L)] for g in range(NG)]
        ses = [SE[l, pl.ds(L * g, L)] for g in range(NG)]

        for r in range(ROWS_PER_W):
            for g in range(NG):
                sl = pl.ds(L * g, L)
                a = A[r, sl]
                b = B[r, sl]
                A[r, sl] = a * ces[g] + b * ses[g]
                B[r, sl] = a * ses[g] - b * ces[g]

        cos_ = [CO[l, pl.ds(L * g, L)] for g in range(NG)]
        sos = [SO[l, pl.ds(L * g, L)] for g in range(NG)]

        for r in range(ROWS_PER_W):
            for g in range(NG):
                sl = pl.ds(L * g, L)
                sl1 = pl.ds(L * g + 1, L)
                b = B[r, sl]
                a1 = A[r, sl1]
                B[r, sl] = b * cos_[g] + a1 * sos[g]
                A[r, sl1] = b * sos[g] - a1 * cos_[g]

    lax.fori_loop(0, NPAIRS // 2, layer_pair2, 0)

    idx_h = iota >> 1
    even_lane = (iota & 1) == 0

    for r in range(ROWS_PER_W):
        for j in range(NG):
            a = A[r, pl.ds(L * j, L)]
            b = B[r, pl.ds(L * j, L)]
            X[r, pl.ds(32 * j, L)] = jnp.where(
                even_lane, _take(a, idx_h), _take(b, idx_h))
            X[r, pl.ds(32 * j + 16, L)] = jnp.where(
                even_lane, _take(a, 8 + idx_h), _take(b, 8 + idx_h))

    pltpu.sync_copy(X, out_hbm.at[pl.ds(base, ROWS_PER_W)])


_clements_sc = pl.kernel(
    _clements_body,
    out_type=jax.ShapeDtypeStruct((SC_ROWS, DIM), jnp.float32),
    mesh=plsc.VectorSubcoreMesh(core_axis_name="c", subcore_axis_name="s",
                                num_cores=2, num_subcores=16),
    compiler_params=pltpu.CompilerParams(use_tc_tiling_on_sc=False),
    scratch_types=[
        pltpu.VMEM((ROWS_PER_W, DIM), jnp.float32),
        pltpu.VMEM((ROWS_PER_W, PAD), jnp.float32),
        pltpu.VMEM((ROWS_PER_W, HALF), jnp.float32),
        pltpu.VMEM((NPAIRS, HALF), jnp.float32),
        pltpu.VMEM((NPAIRS, HALF), jnp.float32),
        pltpu.VMEM((NPAIRS, HALF), jnp.float32),
        pltpu.VMEM((NPAIRS, HALF), jnp.float32),
        pltpu.SemaphoreType.DMA,
    ],
)


def _mm_body(x_ref, w_ref, o_ref):
    o_ref[...] = lax.dot_general(
        x_ref[...], w_ref[...], (((1,), (0,)), ((), ())),
        preferred_element_type=jnp.float32,
        precision=lax.Precision.DEFAULT)


_apply_tc = pl.pallas_call(
    _mm_body,
    grid=(BATCH // MM_TILE,),
    in_specs=[
        pl.BlockSpec((MM_TILE, DIM), lambda i: (i, 0)),
        pl.BlockSpec((DIM, DIM), lambda i: (0, 0)),
    ],
    out_specs=pl.BlockSpec((MM_TILE, DIM), lambda i: (i, 0)),
    out_shape=jax.ShapeDtypeStruct((BATCH, DIM), jnp.float32),
)

def kernel(x, thetas_even, thetas_odd):
    ce, se, co, so = _trig(thetas_even, thetas_odd)
    w = _clements_sc(ce, se, co, so)
    return _apply_tc(x, w)


# speedup vs baseline: 1.0208x; 1.0208x over previous
import jax
import jax.numpy as jnp
from jax import lax
from jax.experimental import pallas as pl
from jax.experimental.pallas import tpu as pltpu
from jax.experimental.pallas import tpu_sc as plsc

DIM = 256
HALF = DIM // 2
PAD = 144
BATCH = 4096
NPAIRS = 64
NW = 32
L = 16
NG = HALF // L
SC_ROWS = DIM
ROWS_PER_W = SC_ROWS // NW
MM_TILE = 2048


def _trig_body(te_ref, to_ref, ce_ref, se_ref, co_ref, so_ref):
    ce_ref[...] = jnp.cos(2.0 * te_ref[...])
    se_ref[...] = jnp.sin(2.0 * te_ref[...])
    to2 = 2.0 * to_ref[...]
    pad1 = jnp.ones((NPAIRS, 1), jnp.float32)
    pad0 = jnp.zeros((NPAIRS, 1), jnp.float32)
    co_ref[...] = jnp.concatenate([jnp.cos(to2), pad1], axis=1)
    so_ref[...] = jnp.concatenate([jnp.sin(to2), pad0], axis=1)


_trig = pl.pallas_call(
    _trig_body,
    out_shape=[jax.ShapeDtypeStruct((NPAIRS, HALF), jnp.float32)] * 4,
)


def _take(v, idx):
    return lax.gather(
        v, idx[:, None],
        lax.GatherDimensionNumbers(offset_dims=(), collapsed_slice_dims=(0,),
                                   start_index_map=(0,)),
        slice_sizes=(1,),
        mode=lax.GatherScatterMode.PROMISE_IN_BOUNDS)


def _clements_body(ce_hbm, se_hbm, co_hbm, so_hbm, out_hbm,
                   X, A, B, CE, SE, CO, SO, dsem):
    wid = lax.axis_index("s") * 2 + lax.axis_index("c")
    base = wid * ROWS_PER_W
    copies = [pltpu.async_copy(src, dst, dsem) for src, dst in
              ((ce_hbm, CE), (se_hbm, SE), (co_hbm, CO), (so_hbm, SO))]

    iota = lax.iota(jnp.int32, L)
    zeros = jnp.zeros((L,), jnp.float32)
    ones = jnp.ones((L,), jnp.float32)

    jc = base // 2
    for r in range(ROWS_PER_W):
        hot = jnp.full((L,), jc + r // 2, jnp.int32)
        for j in range(NG):
            onehot = jnp.where(iota + L * j == hot, ones, zeros)
            if r % 2 == 0:
                A[r, pl.ds(L * j, L)] = onehot
                B[r, pl.ds(L * j, L)] = zeros
            else:
                A[r, pl.ds(L * j, L)] = zeros
                B[r, pl.ds(L * j, L)] = onehot
        A[r, pl.ds(HALF, L)] = zeros

    for c in copies:
        c.wait()

    def layer_pair2(l2, carry):
        for l in (2 * l2, 2 * l2 + 1):
            _do_pair(l)
        return carry

    def _do_pair(l):
        ces = [CE[l, pl.ds(L * g, L)] for g in range(NG)]
        ses = [SE[l, pl.ds(L * g, L)] for g in range(NG)]

        for r in range(ROWS_PER_W):
            for g in range(NG):
                sl = pl.ds(L * g, L)
                a = A[r, sl]
                b = B[r, sl]
                A[r, sl] = a * ces[g] + b * ses[g]
                B[r, sl] = a * ses[g] - b * ces[g]

        cos_ = [CO[l, pl.ds(L * g, L)] for g in range(NG)]
        sos = [SO[l, pl.ds(L * g, L)] for g in range(NG)]

        for r in range(ROWS_PER_W):
            for g in range(NG):
                sl = pl.ds(L * g, L)
                sl1 = pl.ds(L * g + 1, L)
                b = B[r, sl]
                a1 = A[r, sl1]
                B[r, sl] = b * cos_[g] + a1 * sos[g]
                A[r, sl1] = b * sos[g] - a1 * cos_[g]

    lax.fori_loop(0, NPAIRS // 2, layer_pair2, 0)

    idx_h = iota >> 1
    even_lane = (iota & 1) == 0

    for r in range(ROWS_PER_W):
        for j in range(NG):
            a = A[r, pl.ds(L * j, L)]
            b = B[r, pl.ds(L * j, L)]
            X[r, pl.ds(32 * j, L)] = jnp.where(
                even_lane, _take(a, idx_h), _take(b, idx_h))
            X[r, pl.ds(32 * j + 16, L)] = jnp.where(
                even_lane, _take(a, 8 + idx_h), _take(b, 8 + idx_h))

    pltpu.sync_copy(X, out_hbm.at[pl.ds(base, ROWS_PER_W)])


_clements_sc = pl.kernel(
    _clements_body,
    out_type=jax.ShapeDtypeStruct((SC_ROWS, DIM), jnp.float32),
    mesh=plsc.VectorSubcoreMesh(core_axis_name="c", subcore_axis_name="s",
                                num_cores=2, num_subcores=16),
    compiler_params=pltpu.CompilerParams(use_tc_tiling_on_sc=False),
    scratch_types=[
        pltpu.VMEM((ROWS_PER_W, DIM), jnp.float32),
        pltpu.VMEM((ROWS_PER_W, PAD), jnp.float32),
        pltpu.VMEM((ROWS_PER_W, HALF), jnp.float32),
        pltpu.VMEM((NPAIRS, HALF), jnp.float32),
        pltpu.VMEM((NPAIRS, HALF), jnp.float32),
        pltpu.VMEM((NPAIRS, HALF), jnp.float32),
        pltpu.VMEM((NPAIRS, HALF), jnp.float32),
        pltpu.SemaphoreType.DMA,
    ],
)


def _mm_body(x_ref, w_ref, o_ref):
    o_ref[...] = lax.dot_general(
        x_ref[...], w_ref[...], (((1,), (0,)), ((), ())),
        preferred_element_type=jnp.float32,
        precision=lax.Precision.DEFAULT)


_apply_tc = pl.pallas_call(
    _mm_body,
    grid=(BATCH // MM_TILE,),
    in_specs=[
        pl.BlockSpec((MM_TILE, DIM), lambda i: (i, 0)),
        pl.BlockSpec((DIM, DIM), lambda i: (0, 0)),
    ],
    out_specs=pl.BlockSpec((MM_TILE, DIM), lambda i: (i, 0)),
    out_shape=jax.ShapeDtypeStruct((BATCH, DIM), jnp.float32),
)

def kernel(x, thetas_even, thetas_odd):
    ce, se, co, so = _trig(thetas_even, thetas_odd)
    w = _clements_sc(ce, se, co, so)
    return _apply_tc(x, w)
